# baseline (speedup 1.0000x reference)
import functools
import math

import jax
import jax.numpy as jnp
from jax import lax
from jax.experimental import pallas as pl
from jax.experimental.pallas import tpu as pltpu
from jax.experimental.pallas import tpu_sc as plsc

MODEL_DIM = 128
SCALE = math.sqrt(MODEL_DIM)

_INFO = plsc.get_sparse_core_info()
NUM_CORES = _INFO.num_cores
NUM_SUBCORES = _INFO.num_subcores
NUM_WORKERS = NUM_CORES * NUM_SUBCORES

R = 128
NBUF = 4


def _gather_kernel(n_total, d):
    per_worker = n_total // NUM_WORKERS
    n_rounds = per_worker // R
    n_iters = n_rounds // NBUF
    assert per_worker % R == 0 and n_rounds % NBUF == 0

    mesh = plsc.VectorSubcoreMesh(core_axis_name="c", subcore_axis_name="s")

    @functools.partial(
        pl.kernel,
        mesh=mesh,
        out_type=jax.ShapeDtypeStruct((n_total, d), jnp.float32),
        scratch_types=[
            pltpu.VMEM((per_worker,), jnp.int32),
            pltpu.VMEM((NBUF, R, d), jnp.float32),
            pltpu.SemaphoreType.DMA,
            pltpu.SemaphoreType.DMA,
            pltpu.SemaphoreType.DMA,
            pltpu.SemaphoreType.DMA,
            pltpu.SemaphoreType.DMA,
            pltpu.SemaphoreType.DMA,
            pltpu.SemaphoreType.DMA,
            pltpu.SemaphoreType.DMA,
        ],
    )
    def gather(table_hbm, idx_hbm, out_hbm, idx_v, rows_v, *sems):
        gs = sems[:NBUF]
        ws = sems[NBUF:]
        wid = lax.axis_index("s") * NUM_CORES + lax.axis_index("c")
        base = wid * per_worker

        pltpu.sync_copy(idx_hbm.at[pl.ds(base, per_worker)], idx_v)

        def fire_gather(r, b):
            h = R // 2
            pltpu.async_copy(
                table_hbm.at[idx_v.at[pl.ds(r * R, h)]],
                rows_v.at[b, pl.ds(0, h)],
                gs[b],
            )
            pltpu.async_copy(
                table_hbm.at[idx_v.at[pl.ds(r * R + h, h)]],
                rows_v.at[b, pl.ds(h, h)],
                gs[b],
            )

        def wait_gather(b):
            pltpu.make_async_copy(
                table_hbm.at[pl.ds(0, R)], rows_v.at[b], gs[b]
            ).wait()

        def fire_wb(r, b):
            pltpu.async_copy(
                rows_v.at[b], out_hbm.at[pl.ds(base + r * R, R)], ws[b]
            )

        def wait_wb(b):
            pltpu.make_async_copy(
                rows_v.at[b], out_hbm.at[pl.ds(base, R)], ws[b]
            ).wait()

        def scale_buf(b):
            def sbody(rr, c):
                for j in range(d // 16):
                    v = rows_v[b, rr, pl.ds(j * 16, 16)]
                    rows_v[b, rr, pl.ds(j * 16, 16)] = v * SCALE
                return c

            lax.fori_loop(0, R, sbody, 0)

        fire_gather(0, 0)
        fire_gather(1, 1)

        def body(i, carry):
            r0 = NBUF * i

            @pl.when(i > 0)
            def _():
                wait_wb(2)

            fire_gather(r0 + 2, 2)
            wait_gather(0)
            scale_buf(0)
            fire_wb(r0, 0)

            @pl.when(i > 0)
            def _():
                wait_wb(3)

            fire_gather(r0 + 3, 3)
            wait_gather(1)
            scale_buf(1)
            fire_wb(r0 + 1, 1)

            wait_wb(0)

            @pl.when(i + 1 < n_iters)
            def _():
                fire_gather(r0 + 4, 0)

            wait_gather(2)
            scale_buf(2)
            fire_wb(r0 + 2, 2)

            wait_wb(1)

            @pl.when(i + 1 < n_iters)
            def _():
                fire_gather(r0 + 5, 1)

            wait_gather(3)
            scale_buf(3)
            fire_wb(r0 + 3, 3)
            return carry

        lax.fori_loop(0, n_iters, body, 0)
        wait_wb(2)
        wait_wb(3)

    return gather


def kernel(token_ids_batch, embeddings_table):
    batch, seq = token_ids_batch.shape
    _, d = embeddings_table.shape
    n_total = batch * seq

    flat_ids = token_ids_batch.reshape(n_total).astype(jnp.int32)
    out = _gather_kernel(n_total, d)(embeddings_table, flat_ids)
    return out.reshape(batch, seq, d)
